# baseline (speedup 1.0000x reference)
import functools

import jax
import jax.numpy as jnp
from jax.experimental import pallas as pl
from jax.experimental.pallas import tpu as pltpu


def _row_to_col(v_row):
    ones = jnp.ones((1, 1), dtype=v_row.dtype)
    return jax.lax.dot_general(
        v_row, ones,
        dimension_numbers=(((0,), (0,)), ((), ())),
        preferred_element_type=jnp.float32)


def _pass1_kernel(adj_l_ref, adj_r_ref, h_ref, w_ref, u_ref, deg_ref, *, half):
    s = pl.program_id(1)
    al = adj_l_ref[...]
    ar = adj_r_ref[...]
    rowsum = (jnp.sum(al, axis=1, keepdims=True)
              + jnp.sum(ar, axis=1, keepdims=True))
    d_col = jnp.where(rowsum > 0.0,
                      jax.lax.rsqrt(jnp.maximum(rowsum, 1e-30)), 0.0)
    deg_ref[...] = rowsum
    hw = jnp.dot(h_ref[...], w_ref[...],
                 preferred_element_type=jnp.float32)
    dhw = d_col * hw
    dims = (((0,), (0,)), ((), ()))
    cl = jax.lax.dot_general(al, dhw, dimension_numbers=dims,
                             preferred_element_type=jnp.float32)
    cr = jax.lax.dot_general(ar, dhw, dimension_numbers=dims,
                             preferred_element_type=jnp.float32)

    @pl.when(s == 0)
    def _():
        u_ref[0, :half] = cl.astype(jnp.bfloat16)
        u_ref[0, half:] = cr.astype(jnp.bfloat16)

    @pl.when(s > 0)
    def _():
        u_ref[0, :half] = (u_ref[0, :half].astype(jnp.float32)
                           + cl).astype(jnp.bfloat16)
        u_ref[0, half:] = (u_ref[0, half:].astype(jnp.float32)
                           + cr).astype(jnp.bfloat16)


def _gram_kernel(u_ref, deg_ref, b_ref, o_ref, x_scr, *, tm, half_blocks):
    j = pl.program_id(1)

    @pl.when(j == 0)
    def _make_x():
        usum = (u_ref[0].astype(jnp.float32)
                + u_ref[1].astype(jnp.float32))
        deg = deg_ref[...]
        d_col = jnp.where(deg > 0.0,
                          jax.lax.rsqrt(jnp.maximum(deg, 1e-30)), 0.0)
        z = d_col * usum + b_ref[...]
        x_scr[...] = jnp.maximum(z, 0.0).astype(jnp.bfloat16)

    c = pl.program_id(0)
    row = (c * half_blocks + j) * tm
    o_ref[...] = jax.lax.dot_general(
        x_scr[pl.ds(row, tm), :], x_scr[...],
        dimension_numbers=(((1,), (1,)), ((), ())),
        preferred_element_type=jnp.float32)


def kernel(adj, h, w, b):
    N, F = h.shape
    adj = adj.astype(jnp.float32)
    h = h.astype(jnp.float32)
    w = w.astype(jnp.float32)
    b2 = b.reshape(1, F).astype(jnp.float32)

    def pick(tm_want, n):
        tm = min(tm_want, n)
        while n % tm != 0:
            tm //= 2
        return tm

    sub = pick(1024, N // 2)
    S = (N // 2) // sub

    u, deg = pl.pallas_call(
        functools.partial(_pass1_kernel, half=N // 2),
        out_shape=(
            jax.ShapeDtypeStruct((2, N, F), jnp.bfloat16),
            jax.ShapeDtypeStruct((N, 1), jnp.float32),
        ),
        grid=(2, S),
        in_specs=[
            pl.BlockSpec((sub, N // 2), lambda c, s: (c * S + s, 0)),
            pl.BlockSpec((sub, N // 2), lambda c, s: (c * S + s, 1)),
            pl.BlockSpec((sub, F), lambda c, s: (c * S + s, 0)),
            pl.BlockSpec((F, F), lambda c, s: (0, 0)),
        ],
        out_specs=(
            pl.BlockSpec((1, N, F), lambda c, s: (c, 0, 0)),
            pl.BlockSpec((sub, 1), lambda c, s: (c * S + s, 0)),
        ),
        compiler_params=pltpu.CompilerParams(
            dimension_semantics=("parallel", "arbitrary"),
            vmem_limit_bytes=60 << 20,
        ),
    )(adj, adj, h, w)

    tm3 = pick(512, N // 2)
    half_blocks = (N // 2) // tm3

    out = pl.pallas_call(
        functools.partial(_gram_kernel, tm=tm3, half_blocks=half_blocks),
        out_shape=jax.ShapeDtypeStruct((N, N), jnp.float32),
        grid=(2, half_blocks),
        in_specs=[
            pl.BlockSpec((2, N, F), lambda c, j: (0, 0, 0)),
            pl.BlockSpec((N, 1), lambda c, j: (0, 0)),
            pl.BlockSpec((1, F), lambda c, j: (0, 0)),
        ],
        out_specs=pl.BlockSpec(
            (tm3, N),
            lambda c, j, hb=half_blocks: (c * hb + j, 0)),
        scratch_shapes=[
            pltpu.VMEM((N, F), jnp.bfloat16),
        ],
        compiler_params=pltpu.CompilerParams(
            dimension_semantics=("parallel", "arbitrary"),
            vmem_limit_bytes=60 << 20,
        ),
    )(u, deg, b2)

    return out


# speedup vs baseline: 1.0711x; 1.0711x over previous
import functools

import jax
import jax.numpy as jnp
from jax.experimental import pallas as pl
from jax.experimental.pallas import tpu as pltpu


def _row_to_col(v_row):
    ones = jnp.ones((1, 1), dtype=v_row.dtype)
    return jax.lax.dot_general(
        v_row, ones,
        dimension_numbers=(((0,), (0,)), ((), ())),
        preferred_element_type=jnp.float32)


def _pass1_kernel(adj_ref, h_ref, w_ref, u_ref, deg_ref):
    s = pl.program_id(1)
    a = adj_ref[...]
    rowsum = jnp.sum(a, axis=1, keepdims=True)
    d_col = jnp.where(rowsum > 0.0,
                      jax.lax.rsqrt(jnp.maximum(rowsum, 1e-30)), 0.0)
    deg_ref[...] = rowsum
    hw = jnp.dot(h_ref[...], w_ref[...],
                 preferred_element_type=jnp.float32)
    contrib = jax.lax.dot_general(
        a, d_col * hw,
        dimension_numbers=(((0,), (0,)), ((), ())),
        preferred_element_type=jnp.float32)

    @pl.when(s == 0)
    def _():
        u_ref[0] = contrib.astype(jnp.bfloat16)

    @pl.when(s > 0)
    def _():
        u_ref[0] = (u_ref[0].astype(jnp.float32) + contrib).astype(jnp.bfloat16)


def _gram_kernel(u_ref, deg_ref, b_ref, o_ref, x_scr, *, tm, half_blocks):
    j = pl.program_id(1)

    @pl.when(j == 0)
    def _make_x():
        usum = (u_ref[0].astype(jnp.float32)
                + u_ref[1].astype(jnp.float32))
        deg = deg_ref[...]
        d_col = jnp.where(deg > 0.0,
                          jax.lax.rsqrt(jnp.maximum(deg, 1e-30)), 0.0)
        z = d_col * usum + b_ref[...]
        x_scr[...] = jnp.maximum(z, 0.0).astype(jnp.bfloat16)

    c = pl.program_id(0)
    row = (c * half_blocks + j) * tm
    o_ref[...] = jax.lax.dot_general(
        x_scr[pl.ds(row, tm), :], x_scr[...],
        dimension_numbers=(((1,), (1,)), ((), ())),
        preferred_element_type=jnp.float32)


def kernel(adj, h, w, b):
    N, F = h.shape
    adj = adj.astype(jnp.float32)
    h = h.astype(jnp.float32)
    w = w.astype(jnp.float32)
    b2 = b.reshape(1, F).astype(jnp.float32)

    def pick(tm_want, n):
        tm = min(tm_want, n)
        while n % tm != 0:
            tm //= 2
        return tm

    sub = pick(1024, N // 2)
    S = (N // 2) // sub

    u, deg = pl.pallas_call(
        _pass1_kernel,
        out_shape=(
            jax.ShapeDtypeStruct((2, N, F), jnp.bfloat16),
            jax.ShapeDtypeStruct((N, 1), jnp.float32),
        ),
        grid=(2, S),
        in_specs=[
            pl.BlockSpec((sub, N), lambda c, s: (c * S + s, 0)),
            pl.BlockSpec((sub, F), lambda c, s: (c * S + s, 0)),
            pl.BlockSpec((F, F), lambda c, s: (0, 0)),
        ],
        out_specs=(
            pl.BlockSpec((1, N, F), lambda c, s: (c, 0, 0)),
            pl.BlockSpec((sub, 1), lambda c, s: (c * S + s, 0)),
        ),
        compiler_params=pltpu.CompilerParams(
            dimension_semantics=("parallel", "arbitrary"),
            vmem_limit_bytes=60 << 20,
        ),
    )(adj, h, w)

    tm3 = pick(1024, N // 2)
    half_blocks = (N // 2) // tm3

    out = pl.pallas_call(
        functools.partial(_gram_kernel, tm=tm3, half_blocks=half_blocks),
        out_shape=jax.ShapeDtypeStruct((N, N), jnp.float32),
        grid=(2, half_blocks),
        in_specs=[
            pl.BlockSpec((2, N, F), lambda c, j: (0, 0, 0)),
            pl.BlockSpec((N, 1), lambda c, j: (0, 0)),
            pl.BlockSpec((1, F), lambda c, j: (0, 0)),
        ],
        out_specs=pl.BlockSpec(
            (tm3, N),
            lambda c, j, hb=half_blocks: (c * hb + j, 0)),
        scratch_shapes=[
            pltpu.VMEM((N, F), jnp.bfloat16),
        ],
        compiler_params=pltpu.CompilerParams(
            dimension_semantics=("parallel", "arbitrary"),
            vmem_limit_bytes=60 << 20,
        ),
    )(u, deg, b2)

    return out


# speedup vs baseline: 1.0821x; 1.0103x over previous
import functools

import jax
import jax.numpy as jnp
from jax.experimental import pallas as pl
from jax.experimental.pallas import tpu as pltpu


def _row_to_col(v_row):
    ones = jnp.ones((1, 1), dtype=v_row.dtype)
    return jax.lax.dot_general(
        v_row, ones,
        dimension_numbers=(((0,), (0,)), ((), ())),
        preferred_element_type=jnp.float32)


def _pass1_kernel(adj_ref, h_ref, w_ref, u_ref, deg_ref):
    s = pl.program_id(1)
    a = adj_ref[...]
    rowsum = jnp.sum(a, axis=1, keepdims=True)
    d_col = jnp.where(rowsum > 0.0,
                      jax.lax.rsqrt(jnp.maximum(rowsum, 1e-30)), 0.0)
    deg_ref[...] = rowsum
    hw = jnp.dot(h_ref[...], w_ref[...],
                 preferred_element_type=jnp.float32)
    contrib = jax.lax.dot_general(
        a, d_col * hw,
        dimension_numbers=(((0,), (0,)), ((), ())),
        preferred_element_type=jnp.float32)

    @pl.when(s == 0)
    def _():
        u_ref[0] = contrib.astype(jnp.bfloat16)

    @pl.when(s > 0)
    def _():
        u_ref[0] = (u_ref[0].astype(jnp.float32) + contrib).astype(jnp.bfloat16)


def _gram_kernel(u_ref, deg_ref, b_ref, o_ref, x_scr, *, tm, half_blocks):
    j = pl.program_id(1)

    @pl.when(j == 0)
    def _make_x():
        usum = (u_ref[0].astype(jnp.float32)
                + u_ref[1].astype(jnp.float32))
        deg = deg_ref[...]
        d_col = jnp.where(deg > 0.0,
                          jax.lax.rsqrt(jnp.maximum(deg, 1e-30)), 0.0)
        z = d_col * usum + b_ref[...]
        x_scr[...] = jnp.maximum(z, 0.0).astype(jnp.bfloat16)

    c = pl.program_id(0)
    row = (c * half_blocks + j) * tm
    o_ref[...] = jax.lax.dot_general(
        x_scr[pl.ds(row, tm), :], x_scr[...],
        dimension_numbers=(((1,), (1,)), ((), ())),
        preferred_element_type=jnp.float32)


def kernel(adj, h, w, b):
    N, F = h.shape
    adj = adj.astype(jnp.float32)
    h = h.astype(jnp.float32)
    w = w.astype(jnp.float32)
    b2 = b.reshape(1, F).astype(jnp.float32)

    def pick(tm_want, n):
        tm = min(tm_want, n)
        while n % tm != 0:
            tm //= 2
        return tm

    sub = pick(1024, N // 2)
    S = (N // 2) // sub

    u, deg = pl.pallas_call(
        _pass1_kernel,
        out_shape=(
            jax.ShapeDtypeStruct((2, N, F), jnp.bfloat16),
            jax.ShapeDtypeStruct((N, 1), jnp.float32),
        ),
        grid=(2, S),
        in_specs=[
            pl.BlockSpec((sub, N), lambda c, s: (c * S + s, 0)),
            pl.BlockSpec((sub, F), lambda c, s: (c * S + s, 0)),
            pl.BlockSpec((F, F), lambda c, s: (0, 0)),
        ],
        out_specs=(
            pl.BlockSpec((1, N, F), lambda c, s: (c, 0, 0)),
            pl.BlockSpec((sub, 1), lambda c, s: (c * S + s, 0)),
        ),
        compiler_params=pltpu.CompilerParams(
            dimension_semantics=("parallel", "arbitrary"),
            vmem_limit_bytes=60 << 20,
        ),
    )(adj, h, w)

    tm3 = pick(512, N // 2)
    half_blocks = (N // 2) // tm3

    out = pl.pallas_call(
        functools.partial(_gram_kernel, tm=tm3, half_blocks=half_blocks),
        out_shape=jax.ShapeDtypeStruct((N, N), jnp.float32),
        grid=(2, half_blocks),
        in_specs=[
            pl.BlockSpec((2, N, F), lambda c, j: (0, 0, 0)),
            pl.BlockSpec((N, 1), lambda c, j: (0, 0)),
            pl.BlockSpec((1, F), lambda c, j: (0, 0)),
        ],
        out_specs=pl.BlockSpec(
            (tm3, N),
            lambda c, j, hb=half_blocks: (c * hb + j, 0)),
        scratch_shapes=[
            pltpu.VMEM((N, F), jnp.bfloat16),
        ],
        compiler_params=pltpu.CompilerParams(
            dimension_semantics=("parallel", "arbitrary"),
            vmem_limit_bytes=60 << 20,
        ),
    )(u, deg, b2)

    return out
